# baseline (speedup 1.0000x reference)
import jax
import jax.numpy as jnp
from jax import lax
from jax.experimental import pallas as pl
from jax.experimental.pallas import tpu as pltpu
from jax.experimental.pallas import tpu_sc as plsc

NC, NS, LANES = 2, 16, 16
NW = NC * NS
B, L, EMB = 4096, 200, 64
BTOT = B * L
BPW = BTOT // NW
CH = 128
NCH = BPW // CH
SCALE = 8.0

_mesh = plsc.VectorSubcoreMesh(core_axis_name="c", subcore_axis_name="s")


def _emb_body(tok_hbm, table_hbm, out_hbm, idx_v, rows_v, gsem):
    wid = lax.axis_index("s") * NC + lax.axis_index("c")
    base = wid * BPW
    pltpu.sync_copy(tok_hbm.at[wid], idx_v)

    def chunk(g, carry):
        pltpu.async_copy(table_hbm.at[idx_v.at[g]], rows_v, gsem).wait()

        def row(r, c2):
            for j in range(EMB // LANES):
                sl = pl.ds(j * LANES, LANES)
                rows_v[r, sl] = rows_v[r, sl] * SCALE
            return c2

        lax.fori_loop(0, CH, row, 0)
        off = pl.multiple_of(base + g * CH, CH)
        pltpu.sync_copy(rows_v, out_hbm.at[pl.ds(off, CH)])
        return carry

    lax.fori_loop(0, NCH, chunk, 0)


def kernel(tokens, embedding):
    tok = tokens.reshape(NW, NCH, CH)
    out = pl.kernel(
        _emb_body,
        mesh=_mesh,
        compiler_params=pltpu.CompilerParams(use_tc_tiling_on_sc=False),
        out_type=jax.ShapeDtypeStruct((BTOT, EMB), jnp.float32),
        scratch_types=[
            pltpu.VMEM((NCH, CH), jnp.int32),
            pltpu.VMEM((CH, EMB), jnp.float32),
            pltpu.SemaphoreType.DMA,
        ],
    )(tok, embedding)
    return out.reshape(B, L, EMB)


# speedup vs baseline: 1.2067x; 1.2067x over previous
import jax
import jax.numpy as jnp
from jax import lax
from jax.experimental import pallas as pl
from jax.experimental.pallas import tpu as pltpu
from jax.experimental.pallas import tpu_sc as plsc

NC, NS, LANES = 2, 16, 16
NW = NC * NS
B, L, EMB = 4096, 200, 64
BTOT = B * L
BPW = BTOT // NW
CH = 128
NCH = BPW // CH
NB = 8
NITER = NCH // NB
RU = 8
SCALE = 8.0

_mesh = plsc.VectorSubcoreMesh(core_axis_name="c", subcore_axis_name="s")


def _emb_body(tok_hbm, table_hbm, out_hbm, idx_v, *scratch):
    rows = scratch[:NB]
    gsems = scratch[NB:2 * NB]
    wsems = scratch[2 * NB:3 * NB]
    wid = lax.axis_index("s") * NC + lax.axis_index("c")
    base = wid * BPW
    pltpu.sync_copy(tok_hbm.at[wid], idx_v)

    for b in range(NB):
        pltpu.async_copy(table_hbm.at[idx_v.at[b]], rows[b], gsems[b])

    def scale_buf(buf):
        def body(r, c):
            rb = r * RU
            for u in range(RU):
                for j in range(EMB // LANES):
                    sl = pl.ds(j * LANES, LANES)
                    buf[rb + u, sl] = buf[rb + u, sl] * SCALE
            return c

        lax.fori_loop(0, CH // RU, body, 0)

    def outer(it, carry):
        g0 = it * NB
        for b in range(NB):
            g = g0 + b
            pltpu.make_async_copy(
                table_hbm.at[idx_v.at[g]], rows[b], gsems[b]).wait()
            scale_buf(rows[b])
            off = pl.multiple_of(base + g * CH, CH)
            pltpu.async_copy(rows[b], out_hbm.at[pl.ds(off, CH)], wsems[b])

            @pl.when(g + NB < NCH)
            def _refire():
                pltpu.make_async_copy(
                    rows[b], out_hbm.at[pl.ds(off, CH)], wsems[b]).wait()
                pltpu.async_copy(
                    table_hbm.at[idx_v.at[g + NB]], rows[b], gsems[b])

        return carry

    lax.fori_loop(0, NITER, outer, 0)

    for b in range(NB):
        pltpu.make_async_copy(
            rows[b], out_hbm.at[pl.ds(base, CH)], wsems[b]).wait()


def kernel(tokens, embedding):
    tok = tokens.reshape(NW, NCH, CH)
    out = pl.kernel(
        _emb_body,
        mesh=_mesh,
        compiler_params=pltpu.CompilerParams(use_tc_tiling_on_sc=False),
        out_type=jax.ShapeDtypeStruct((BTOT, EMB), jnp.float32),
        scratch_types=(
            [pltpu.VMEM((NCH, CH), jnp.int32)]
            + [pltpu.VMEM((CH, EMB), jnp.float32) for _ in range(NB)]
            + [pltpu.SemaphoreType.DMA for _ in range(2 * NB)]
        ),
    )(tok, embedding)
    return out.reshape(B, L, EMB)


# speedup vs baseline: 1.2088x; 1.0017x over previous
import jax
import jax.numpy as jnp
from jax import lax
from jax.experimental import pallas as pl
from jax.experimental.pallas import tpu as pltpu
from jax.experimental.pallas import tpu_sc as plsc

NC, NS, LANES = 2, 16, 16
NW = NC * NS
B, L, EMB = 4096, 200, 64
BTOT = B * L
BPW = BTOT // NW
CH = 128
NCH = BPW // CH
NB = 8
NITER = NCH // NB
RU = 8
SCALE = 8.0

_mesh = plsc.VectorSubcoreMesh(core_axis_name="c", subcore_axis_name="s")


def _emb_body(tok_hbm, table_hbm, out_hbm, idx_v, *scratch):
    rows = scratch[:NB]
    gsems = scratch[NB:2 * NB]
    wsems = scratch[2 * NB:3 * NB]
    wid = lax.axis_index("s") * NC + lax.axis_index("c")
    base = wid * BPW
    pltpu.sync_copy(tok_hbm.at[wid], idx_v)

    for b in range(NB):
        pltpu.async_copy(table_hbm.at[idx_v.at[b]], rows[b], gsems[b])

    def scale_buf(buf):
        @plsc.parallel_loop(0, CH, 1, unroll=RU)
        def _scale(r):
            for j in range(EMB // LANES):
                sl = pl.ds(j * LANES, LANES)
                buf[r, sl] = buf[r, sl] * SCALE

    def outer(it, carry):
        g0 = it * NB
        for b in range(NB):
            g = g0 + b
            pltpu.make_async_copy(
                table_hbm.at[idx_v.at[g]], rows[b], gsems[b]).wait()
            scale_buf(rows[b])
            off = pl.multiple_of(base + g * CH, CH)
            pltpu.async_copy(rows[b], out_hbm.at[pl.ds(off, CH)], wsems[b])

            @pl.when(g + NB < NCH)
            def _refire():
                pltpu.make_async_copy(
                    rows[b], out_hbm.at[pl.ds(off, CH)], wsems[b]).wait()
                pltpu.async_copy(
                    table_hbm.at[idx_v.at[g + NB]], rows[b], gsems[b])

        return carry

    lax.fori_loop(0, NITER, outer, 0)

    for b in range(NB):
        pltpu.make_async_copy(
            rows[b], out_hbm.at[pl.ds(base, CH)], wsems[b]).wait()


def kernel(tokens, embedding):
    tok = tokens.reshape(NW, NCH, CH)
    out = pl.kernel(
        _emb_body,
        mesh=_mesh,
        compiler_params=pltpu.CompilerParams(use_tc_tiling_on_sc=False),
        out_type=jax.ShapeDtypeStruct((BTOT, EMB), jnp.float32),
        scratch_types=(
            [pltpu.VMEM((NCH, CH), jnp.int32)]
            + [pltpu.VMEM((CH, EMB), jnp.float32) for _ in range(NB)]
            + [pltpu.SemaphoreType.DMA for _ in range(2 * NB)]
        ),
    )(tok, embedding)
    return out.reshape(B, L, EMB)
